# Initial kernel scaffold; baseline (speedup 1.0000x reference)
import jax
import jax.numpy as jnp
from jax.experimental import pallas as pl


def kernel(x, edge_index, edge_attr, batch, W1, b1, Wc1, Wc2, Wc3, Wc4, W2, b2, Wg, bg, Wt, bt):
    raise NotImplementedError("write your pallas kernel here")


# speedup vs baseline: 6.9985x; 6.9985x over previous
import functools

import jax
import jax.numpy as jnp
from jax import lax
from jax.experimental import pallas as pl
from jax.experimental.pallas import tpu as pltpu
from jax.experimental.pallas import tpu_sc as plsc

_N = 10000
_NP = 10240
_E = 320000
_H = 256
_HH = 128
_G = 256
_ALPHA = 0.2
_EPS = 1e-5

_NC = 2
_NS = 16
_K = 80
_RPT = _NP // _NS

_EPC = _E // _NC
_EPT_D = _EPC // _NS
_NCH_D = _EPT_D // _K

_EPT_P = _E // _NS
_NCH_P = _EPT_P // _K

_BM = 2048
_NBLK = _NP // _BM


@functools.cache
def _sc_mesh():
    return plsc.VectorSubcoreMesh(core_axis_name="c", subcore_axis_name="s")


def _deg_body(col_hbm, ones_hbm, zeros_hbm, out_hbm, acc_sp, colbuf, onesbuf):
    c = lax.axis_index("c")
    s = lax.axis_index("s")
    for j in range(_RPT // 128):
        pltpu.sync_copy(zeros_hbm, acc_sp.at[pl.ds(s * _RPT + j * 128, 128)])
    pltpu.sync_copy(ones_hbm, onesbuf)
    plsc.subcore_barrier()
    base = c * _EPC + s * _EPT_D

    def body(i, carry):
        pltpu.sync_copy(col_hbm.at[pl.ds(base + i * _K, _K)], colbuf)
        pltpu.sync_copy(onesbuf, acc_sp.at[colbuf], add=True)
        return carry

    lax.fori_loop(0, _NCH_D, body, 0)
    plsc.subcore_barrier()
    pltpu.sync_copy(acc_sp.at[pl.ds(s * _RPT, _RPT)],
                    out_hbm.at[c, pl.ds(s * _RPT, _RPT)])


@functools.cache
def _deg_call():
    return pl.kernel(
        _deg_body,
        out_type=jax.ShapeDtypeStruct((_NC, _NP, _HH), jnp.float32),
        mesh=_sc_mesh(),
        scratch_types=[
            pltpu.VMEM_SHARED((_NP, _HH), jnp.float32),
            pltpu.VMEM((_K,), jnp.int32),
            pltpu.VMEM((_K, _HH), jnp.float32),
        ],
    )


def _prop_body(zlo_hbm, zhi_hbm, row_hbm, col_hbm,
               ulo_hbm, uhi_hbm, acc_sp, rowbuf, colbuf, gbuf, sem):
    c = lax.axis_index("c")
    s = lax.axis_index("s")
    base = s * _EPT_P

    def run(table_hbm, out_hbm):
        pltpu.sync_copy(table_hbm.at[pl.ds(s * _RPT, _RPT)],
                        acc_sp.at[pl.ds(s * _RPT, _RPT)])
        plsc.subcore_barrier()

        def body(i, carry):
            pltpu.sync_copy(row_hbm.at[pl.ds(base + i * _K, _K)], rowbuf)
            pltpu.sync_copy(col_hbm.at[pl.ds(base + i * _K, _K)], colbuf)
            pltpu.async_copy(table_hbm.at[rowbuf], gbuf, sem).wait()
            pltpu.sync_copy(gbuf, acc_sp.at[colbuf], add=True)
            return carry

        lax.fori_loop(0, _NCH_P, body, 0)
        plsc.subcore_barrier()
        pltpu.sync_copy(acc_sp.at[pl.ds(s * _RPT, _RPT)],
                        out_hbm.at[pl.ds(s * _RPT, _RPT)])

    @pl.when(c == 0)
    def _():
        run(zlo_hbm, ulo_hbm)

    @pl.when(c == 1)
    def _():
        run(zhi_hbm, uhi_hbm)


@functools.cache
def _prop_call():
    return pl.kernel(
        _prop_body,
        out_type=(jax.ShapeDtypeStruct((_NP, _HH), jnp.float32),
                  jax.ShapeDtypeStruct((_NP, _HH), jnp.float32)),
        mesh=_sc_mesh(),
        scratch_types=[
            pltpu.VMEM_SHARED((_NP, _HH), jnp.float32),
            pltpu.VMEM((_K,), jnp.int32),
            pltpu.VMEM((_K,), jnp.int32),
            pltpu.VMEM((_K, _HH), jnp.float32),
            pltpu.SemaphoreType.DMA,
        ],
    )


def _pad_mask(vals, pid):
    rid = lax.broadcasted_iota(jnp.int32, vals.shape, 0) + pid * _BM
    return jnp.where(rid < _N, vals, 0.0)


def _k0_body(x_ref, w1_ref, b1_ref, degp_ref,
             x0_ref, dis_ref, zlo_ref, zhi_ref):
    pid = pl.program_id(0)
    x0 = jnp.dot(x_ref[...], w1_ref[...],
                 preferred_element_type=jnp.float32) + b1_ref[...]
    x0 = _pad_mask(x0, pid)
    deg = (degp_ref[0, :, 0:1] + degp_ref[1, :, 0:1]) + 1.0
    dis = lax.rsqrt(deg)
    z = _pad_mask(dis * x0, pid)
    x0_ref[...] = x0
    dis_ref[...] = dis
    zlo_ref[...] = z[:, :_HH]
    zhi_ref[...] = z[:, _HH:]


_k0_call = pl.pallas_call(
    _k0_body,
    grid=(_NBLK,),
    in_specs=[
        pl.BlockSpec((_BM, 128), lambda i: (i, 0)),
        pl.BlockSpec((128, _H), lambda i: (0, 0)),
        pl.BlockSpec((1, _H), lambda i: (0, 0)),
        pl.BlockSpec((_NC, _BM, _HH), lambda i: (0, i, 0)),
    ],
    out_specs=(
        pl.BlockSpec((_BM, _H), lambda i: (i, 0)),
        pl.BlockSpec((_BM, 1), lambda i: (i, 0)),
        pl.BlockSpec((_BM, _HH), lambda i: (i, 0)),
        pl.BlockSpec((_BM, _HH), lambda i: (i, 0)),
    ),
    out_shape=(jax.ShapeDtypeStruct((_NP, _H), jnp.float32),
               jax.ShapeDtypeStruct((_NP, 1), jnp.float32),
               jax.ShapeDtypeStruct((_NP, _HH), jnp.float32),
               jax.ShapeDtypeStruct((_NP, _HH), jnp.float32)),
)


def _stage_a_body(ulo_ref, uhi_ref, dis_ref, x0_ref, wc_ref,
                  t_ref, stats_ref, acc):
    pid = pl.program_id(0)
    u = jnp.concatenate([ulo_ref[...], uhi_ref[...]], axis=1)
    m = (1.0 - _ALPHA) * (dis_ref[...] * u) + _ALPHA * x0_ref[...]
    t = jnp.dot(m, wc_ref[...], preferred_element_type=jnp.float32)
    t_ref[...] = t

    @pl.when(pid == 0)
    def _():
        acc[...] = jnp.zeros_like(acc)

    s1 = jnp.sum(t, axis=0, keepdims=True)
    s2 = jnp.sum(t * t, axis=0, keepdims=True)
    acc[0:1, :] += s1
    acc[1:2, :] += s2
    stats_ref[...] = acc[...]


_stage_a_call = pl.pallas_call(
    _stage_a_body,
    grid=(_NBLK,),
    in_specs=[
        pl.BlockSpec((_BM, _HH), lambda i: (i, 0)),
        pl.BlockSpec((_BM, _HH), lambda i: (i, 0)),
        pl.BlockSpec((_BM, 1), lambda i: (i, 0)),
        pl.BlockSpec((_BM, _H), lambda i: (i, 0)),
        pl.BlockSpec((_H, _H), lambda i: (0, 0)),
    ],
    out_specs=(
        pl.BlockSpec((_BM, _H), lambda i: (i, 0)),
        pl.BlockSpec((8, _H), lambda i: (0, 0)),
    ),
    out_shape=(jax.ShapeDtypeStruct((_NP, _H), jnp.float32),
               jax.ShapeDtypeStruct((8, _H), jnp.float32)),
    scratch_shapes=[pltpu.VMEM((8, _H), jnp.float32)],
)


def _stage_b_body(has_res, has_h, has_z, *refs):
    i = 0
    t_ref = refs[i]; i += 1
    stats_ref = refs[i]; i += 1
    dis_ref = refs[i]; i += 1
    if has_res:
        res_ref = refs[i]; i += 1
    outs = refs[i:]
    pid = pl.program_id(0)
    mean = stats_ref[0:1, :] * (1.0 / _N)
    var = stats_ref[1:2, :] * (1.0 / _N) - mean * mean
    tn = (t_ref[...] - mean) / jnp.sqrt(var + _EPS)
    if has_res:
        tn = tn + res_ref[...]
    h = _pad_mask(jnp.maximum(tn, 0.0), pid)
    oi = 0
    if has_h:
        outs[oi][...] = h
        oi += 1
    if has_z:
        z = dis_ref[...] * h
        outs[oi][...] = z[:, :_HH]
        outs[oi + 1][...] = z[:, _HH:]


def _make_stage_b(has_res, has_h, has_z):
    in_specs = [
        pl.BlockSpec((_BM, _H), lambda i: (i, 0)),
        pl.BlockSpec((8, _H), lambda i: (0, 0)),
        pl.BlockSpec((_BM, 1), lambda i: (i, 0)),
    ]
    if has_res:
        in_specs.append(pl.BlockSpec((_BM, _H), lambda i: (i, 0)))
    out_specs, out_shape = [], []
    if has_h:
        out_specs.append(pl.BlockSpec((_BM, _H), lambda i: (i, 0)))
        out_shape.append(jax.ShapeDtypeStruct((_NP, _H), jnp.float32))
    if has_z:
        out_specs += [pl.BlockSpec((_BM, _HH), lambda i: (i, 0)),
                      pl.BlockSpec((_BM, _HH), lambda i: (i, 0))]
        out_shape += [jax.ShapeDtypeStruct((_NP, _HH), jnp.float32),
                      jax.ShapeDtypeStruct((_NP, _HH), jnp.float32)]
    return pl.pallas_call(
        functools.partial(_stage_b_body, has_res, has_h, has_z),
        grid=(_NBLK,),
        in_specs=in_specs,
        out_specs=tuple(out_specs),
        out_shape=tuple(out_shape),
    )


_stage_b_l1 = _make_stage_b(False, True, True)
_stage_b_l2 = _make_stage_b(True, True, True)
_stage_b_l3 = _make_stage_b(False, False, True)
_stage_b_l4 = _make_stage_b(True, True, False)


def _head_body(h4_ref, w2_ref, b2_ref, wg_ref, bg_ref, wt_ref, bt_ref,
               ae_ref, ap_ref, gate_ref, theta_ref):
    h4 = h4_ref[...]
    nrm = jnp.sqrt(jnp.sum(h4 * h4, axis=1, keepdims=True))
    ae_ref[...] = h4 / jnp.maximum(nrm, 1e-12)
    ap_ref[...] = jnp.dot(h4, w2_ref[...],
                          preferred_element_type=jnp.float32) + b2_ref[...]
    gate_ref[...] = jnp.dot(h4, wg_ref[...],
                            preferred_element_type=jnp.float32) + bg_ref[...]
    theta_ref[...] = jnp.dot(h4, wt_ref[...],
                             preferred_element_type=jnp.float32) + bt_ref[...]


_head_call = pl.pallas_call(
    _head_body,
    grid=(_N // 2000,),
    in_specs=[
        pl.BlockSpec((2000, _H), lambda i: (i, 0)),
        pl.BlockSpec((_H, 128), lambda i: (0, 0)),
        pl.BlockSpec((1, 128), lambda i: (0, 0)),
        pl.BlockSpec((_H, 1), lambda i: (0, 0)),
        pl.BlockSpec((1, 1), lambda i: (0, 0)),
        pl.BlockSpec((_H, _H), lambda i: (0, 0)),
        pl.BlockSpec((1, _H), lambda i: (0, 0)),
    ],
    out_specs=(
        pl.BlockSpec((2000, _H), lambda i: (i, 0)),
        pl.BlockSpec((2000, 128), lambda i: (i, 0)),
        pl.BlockSpec((2000, 1), lambda i: (i, 0)),
        pl.BlockSpec((2000, _H), lambda i: (i, 0)),
    ),
    out_shape=(jax.ShapeDtypeStruct((_N, _H), jnp.float32),
               jax.ShapeDtypeStruct((_N, 128), jnp.float32),
               jax.ShapeDtypeStruct((_N, 1), jnp.float32),
               jax.ShapeDtypeStruct((_N, _H), jnp.float32)),
)

_BP = 2000


def _pool1_body(gate_ref, batch_ref, gm_ref, acc):
    pid = pl.program_id(0)

    @pl.when(pid == 0)
    def _():
        acc[0:1, :] = jnp.full((1, _G), -jnp.inf, jnp.float32)
        acc[1:2, :] = jnp.zeros((1, _G), jnp.float32)

    gids = lax.broadcasted_iota(jnp.int32, (1, _G), 1)
    P = (batch_ref[...] == gids).astype(jnp.float32)
    blkmax = jnp.max(jnp.where(P > 0, gate_ref[...], -jnp.inf),
                     axis=0, keepdims=True)
    acc[0:1, :] = jnp.maximum(acc[0:1, :], blkmax)
    acc[1:2, :] += jnp.sum(P, axis=0, keepdims=True)
    gm = jnp.where(acc[1:2, :] > 0, acc[0:1, :], 0.0)
    gm_ref[0:1, :] = gm
    gm_ref[1:8, :] = jnp.zeros((7, _G), jnp.float32)


_pool1_call = pl.pallas_call(
    _pool1_body,
    grid=(_N // _BP,),
    in_specs=[
        pl.BlockSpec((_BP, 1), lambda i: (i, 0)),
        pl.BlockSpec((_BP, 1), lambda i: (i, 0)),
    ],
    out_specs=pl.BlockSpec((8, _G), lambda i: (0, 0)),
    out_shape=jax.ShapeDtypeStruct((8, _G), jnp.float32),
    scratch_shapes=[pltpu.VMEM((8, _G), jnp.float32)],
)


def _pool2_body(gate_ref, batch_ref, gm_ref, e_ref, sg_ref, acc):
    pid = pl.program_id(0)

    @pl.when(pid == 0)
    def _():
        acc[...] = jnp.zeros_like(acc)

    gids = lax.broadcasted_iota(jnp.int32, (1, _G), 1)
    P = (batch_ref[...] == gids).astype(jnp.float32)
    gmn = jnp.sum(P * gm_ref[0:1, :], axis=1, keepdims=True)
    e = jnp.exp(gate_ref[...] - gmn)
    e_ref[...] = e
    acc[0:1, :] += jnp.sum(P * e, axis=0, keepdims=True)
    sg_ref[0:1, :] = acc[0:1, :]
    sg_ref[1:8, :] = jnp.zeros((7, _G), jnp.float32)


_pool2_call = pl.pallas_call(
    _pool2_body,
    grid=(_N // _BP,),
    in_specs=[
        pl.BlockSpec((_BP, 1), lambda i: (i, 0)),
        pl.BlockSpec((_BP, 1), lambda i: (i, 0)),
        pl.BlockSpec((8, _G), lambda i: (0, 0)),
    ],
    out_specs=(
        pl.BlockSpec((_BP, 1), lambda i: (i, 0)),
        pl.BlockSpec((8, _G), lambda i: (0, 0)),
    ),
    out_shape=(jax.ShapeDtypeStruct((_N, 1), jnp.float32),
               jax.ShapeDtypeStruct((8, _G), jnp.float32)),
    scratch_shapes=[pltpu.VMEM((8, _G), jnp.float32)],
)


def _pool3_body(e_ref, batch_ref, sg_ref, theta_ref, mol_ref, acc):
    pid = pl.program_id(0)

    @pl.when(pid == 0)
    def _():
        acc[...] = jnp.zeros_like(acc)

    gids = lax.broadcasted_iota(jnp.int32, (1, _G), 1)
    P = (batch_ref[...] == gids).astype(jnp.float32)
    sn = jnp.sum(P * sg_ref[0:1, :], axis=1, keepdims=True)
    attn = e_ref[...] / (sn + 1e-16)
    acc[...] += lax.dot_general(
        P, attn * theta_ref[...],
        dimension_numbers=(((0,), (0,)), ((), ())),
        preferred_element_type=jnp.float32)
    mol_ref[...] = acc[...]


_pool3_call = pl.pallas_call(
    _pool3_body,
    grid=(_N // _BP,),
    in_specs=[
        pl.BlockSpec((_BP, 1), lambda i: (i, 0)),
        pl.BlockSpec((_BP, 1), lambda i: (i, 0)),
        pl.BlockSpec((8, _G), lambda i: (0, 0)),
        pl.BlockSpec((_BP, _H), lambda i: (i, 0)),
    ],
    out_specs=pl.BlockSpec((_G, _H), lambda i: (0, 0)),
    out_shape=jax.ShapeDtypeStruct((_G, _H), jnp.float32),
    scratch_shapes=[pltpu.VMEM((_G, _H), jnp.float32)],
)


def kernel(x, edge_index, edge_attr, batch,
           W1, b1, Wc1, Wc2, Wc3, Wc4, W2, b2, Wg, bg, Wt, bt):
    row = edge_index[0]
    col = edge_index[1]
    ones_sc = jnp.ones((_K, _HH), jnp.float32)
    zeros_sc = jnp.zeros((128, _HH), jnp.float32)
    xp = jnp.pad(x, ((0, _NP - _N), (0, 0)))

    deg_call = _deg_call()
    prop_call = _prop_call()
    degp = deg_call(col, ones_sc, zeros_sc)
    x0, dis, zlo, zhi = _k0_call(xp, W1, b1.reshape(1, -1), degp)

    ulo, uhi = prop_call(zlo, zhi, row, col)
    t, st = _stage_a_call(ulo, uhi, dis, x0, Wc1)
    h1, zlo, zhi = _stage_b_l1(t, st, dis)
    ulo, uhi = prop_call(zlo, zhi, row, col)
    t, st = _stage_a_call(ulo, uhi, dis, x0, Wc2)
    h2, zlo, zhi = _stage_b_l2(t, st, dis, h1)
    ulo, uhi = prop_call(zlo, zhi, row, col)
    t, st = _stage_a_call(ulo, uhi, dis, x0, Wc3)
    (zlo, zhi) = _stage_b_l3(t, st, dis)
    ulo, uhi = prop_call(zlo, zhi, row, col)
    t, st = _stage_a_call(ulo, uhi, dis, x0, Wc4)
    (h4,) = _stage_b_l4(t, st, dis, h2)

    ae, ap, gate, theta = _head_call(h4, W2, b2.reshape(1, -1),
                                     Wg, bg.reshape(1, -1),
                                     Wt, bt.reshape(1, -1))
    batch2 = batch.reshape(-1, 1)
    gm = _pool1_call(gate, batch2)
    e, sg = _pool2_call(gate, batch2, gm)
    mol = _pool3_call(e, batch2, sg, theta)
    return (ae, ap, mol)
